# baseline (speedup 1.0000x reference)
import functools

import jax
import jax.numpy as jnp
from jax import lax
from jax.experimental import pallas as pl
from jax.experimental.pallas import tpu as pltpu
from jax.experimental.pallas import tpu_sc as plsc

VOCAB = 1000000
D = 64
N_TOK = 20
B = 1024
S = 200
OUT_S = N_TOK + S

NC = 2
NS = 16
NBG = 8
NSQ = 4
SQ = S // NSQ
LANES = 128


def _body(tokT_hbm, wte2_hbm, learned_hbm, out_hbm,
          tokv, idx_v, gbuf, obuf, lbuf, gsem):
    wid = lax.axis_index("s") * NC + lax.axis_index("c")
    bg = wid % NBG
    sq = wid // NBG
    lane0 = bg * LANES

    pltpu.sync_copy(tokT_hbm.at[:, pl.ds(lane0, LANES)], tokv)
    pltpu.sync_copy(learned_hbm, lbuf)

    iota16 = lax.iota(jnp.int32, 16)

    @pl.when(sq == 0)
    def _prompt():
        def t_body(t, carry):
            def d_body(d, c2):
                v = plsc.load_gather(
                    lbuf,
                    [jnp.full((16,), t, jnp.int32),
                     jnp.full((16,), d, jnp.int32)],
                )
                for c in range(8):
                    plsc.store_scatter(
                        obuf,
                        [jnp.full((16,), d, jnp.int32), iota16 + (16 * c)],
                        v,
                    )
                return c2
            lax.fori_loop(0, D, d_body, 0)
            pltpu.sync_copy(obuf, out_hbm.at[t, :, pl.ds(lane0, LANES)])
            return carry
        lax.fori_loop(0, N_TOK, t_body, 0)

    def s_body(s, carry):
        for c in range(8):
            tk = tokv[s, pl.ds(16 * c, 16)]
            idx_v[pl.ds(16 * c, 16)] = lax.shift_right_logical(tk, 1)
        pltpu.async_copy(wte2_hbm.at[idx_v], gbuf, gsem).wait()
        for c in range(8):
            tk = tokv[s, pl.ds(16 * c, 16)]
            rows = iota16 + (16 * c)
            colbase = lax.shift_left(lax.bitwise_and(tk, 1), 6)
            def d_body(d, c2):
                for du in range(4):
                    vals = plsc.load_gather(gbuf, [rows, colbase + (d * 4 + du)])
                    plsc.store_scatter(
                        obuf,
                        [jnp.full((16,), d * 4 + du, jnp.int32), rows],
                        vals,
                    )
                return c2
            lax.fori_loop(0, D // 4, d_body, 0)
        pltpu.sync_copy(obuf, out_hbm.at[s + N_TOK, :, pl.ds(lane0, LANES)])
        return carry

    lax.fori_loop(SQ * sq, SQ * sq + SQ, s_body, 0)


@functools.partial(jax.jit)
def kernel(tokens, wte_weight, learned_embedding):
    tokT = tokens.T.astype(jnp.int32)
    wte2 = wte_weight.reshape(VOCAB // 2, 2 * D)
    mesh = plsc.VectorSubcoreMesh(core_axis_name="c", subcore_axis_name="s")
    k = pl.kernel(
        _body,
        mesh=mesh,
        out_type=jax.ShapeDtypeStruct((OUT_S, D, B), jnp.float32),
        scratch_types=[
            pltpu.VMEM((S, LANES), jnp.int32),
            pltpu.VMEM((LANES,), jnp.int32),
            pltpu.VMEM((LANES, 2 * D), jnp.float32),
            pltpu.VMEM((D, LANES), jnp.float32),
            pltpu.VMEM((N_TOK, D), jnp.float32),
            pltpu.SemaphoreType.DMA,
        ],
        compiler_params=pltpu.CompilerParams(
            use_tc_tiling_on_sc=True, needs_layout_passes=False),
    )
    out3 = k(tokT, wte2, learned_embedding)
    return out3.transpose(2, 0, 1)
